# baseline (speedup 1.0000x reference)
import jax
import jax.numpy as jnp
from jax import lax
from jax.experimental import pallas as pl
from jax.experimental.pallas import tpu as pltpu
from jax.experimental.pallas import tpu_sc as plsc

N = 16384
C = 1000
NUM_PROB_BINS = 10
ALPHA = 0.99
ROWS = 2048
NB = N // ROWS
PB_PAD = 128

NW = 32
RW = N // NW
RQ = RW // 128
C_PAD = 1024


def _sc_body(pred_hbm, tgt_hbm, ce_hbm, zeros_hbm, ones_hbm,
             tval_hbm, cet_hbm, cnt_hbm,
             tgt_v, idx_v, tval_v, cet_v, ones_v, cnt_s, sem):
    sid = lax.axis_index("s")
    cid = lax.axis_index("c")
    wid = sid * 2 + cid
    base = wid * RW
    pltpu.sync_copy(tgt_hbm.at[wid], tgt_v)
    pltpu.sync_copy(ones_hbm, ones_v)

    @pl.when(sid == 0)
    def _zero_shared():
        pltpu.sync_copy(zeros_hbm, cnt_s)

    l16 = lax.iota(jnp.int32, 16)
    for r in range(RQ):
        for k in range(8):
            t16 = tgt_v[r, pl.ds(k * 16, 16)]
            row0 = base + r * 128 + k * 16
            idx_v[r, pl.ds(k * 16, 16)] = (l16 + row0) * C + t16
    copies = [pltpu.async_copy(pred_hbm.at[idx_v.at[r]], tval_v.at[r], sem)
              for r in range(RQ)]
    copies += [pltpu.async_copy(ce_hbm.at[tgt_v.at[r]], cet_v.at[r], sem)
               for r in range(RQ)]
    for cp in copies:
        cp.wait()

    plsc.subcore_barrier()
    for r in range(RQ):
        pltpu.sync_copy(ones_v.at[r], cnt_s.at[tgt_v.at[r]], add=True)
    plsc.subcore_barrier()

    @pl.when(sid == 0)
    def _write_shared():
        pltpu.sync_copy(cnt_s, cnt_hbm.at[cid])

    pltpu.sync_copy(tval_v, tval_hbm.at[wid])
    pltpu.sync_copy(cet_v, cet_hbm.at[wid])


_sc_gather = pl.kernel(
    _sc_body,
    mesh=plsc.VectorSubcoreMesh(core_axis_name="c", subcore_axis_name="s"),
    out_type=[
        jax.ShapeDtypeStruct((NW, RQ, 128), jnp.float32),
        jax.ShapeDtypeStruct((NW, RQ, 128), jnp.float32),
        jax.ShapeDtypeStruct((2, C_PAD), jnp.float32),
    ],
    scratch_types=[
        pltpu.VMEM((RQ, 128), jnp.int32),
        pltpu.VMEM((RQ, 128), jnp.int32),
        pltpu.VMEM((RQ, 128), jnp.float32),
        pltpu.VMEM((RQ, 128), jnp.float32),
        pltpu.VMEM((RQ, 128), jnp.float32),
        pltpu.VMEM_SHARED((C_PAD,), jnp.float32),
        pltpu.SemaphoreType.DMA,
    ],
)


def _rowstats_body(pred_ref, m_out, s_out):
    x = pred_ref[...]
    m = jnp.max(x, axis=1, keepdims=True)
    s = jnp.sum(jnp.exp(x - m), axis=1, keepdims=True)
    m_out[...] = m
    s_out[...] = s


def _final_body(m_ref, s_ref, tv_ref, cet_ref, cnt_ref, ce_ref, pb_ref,
                loss_out, pb_out, cls_out):
    m = m_ref[...]
    s = s_ref[...]
    tv = tv_ref[...]
    cet = cet_ref[...]

    lse = m + jnp.log(s)
    loss = lse - tv
    p = jnp.clip(jnp.exp(tv - m) / s, 1e-06, 1.0 - 1e-06)

    bin_idx = jnp.clip(jnp.floor(p * NUM_PROB_BINS - 1e-06).astype(jnp.int32),
                       0, NUM_PROB_BINS - 1)
    hist_idx = jnp.clip(jnp.floor(p * NUM_PROB_BINS).astype(jnp.int32),
                        0, NUM_PROB_BINS - 1)

    pb = pb_ref[...]
    lane_iota = lax.broadcasted_iota(jnp.int32, (1, PB_PAD), 1)
    pb_t = jnp.zeros_like(p)
    hist = jnp.zeros((1, PB_PAD), jnp.float32)
    for b in range(NUM_PROB_BINS):
        pb_b = pb[0:1, b:b + 1]
        pb_t = pb_t + jnp.where(bin_idx == b, pb_b, 0.0)
        cnt_b = jnp.sum((hist_idx == b).astype(jnp.float32))
        hist = hist + cnt_b * (lane_iota == b).astype(jnp.float32)

    w = jnp.sqrt(cet * pb_t + 1e-10)
    loss_out[...] = jnp.full((1, PB_PAD), jnp.sum(loss / w) / N)

    prob_bins = hist / (jnp.sum(hist) + 1e-10) * NUM_PROB_BINS
    new_pb = pb * ALPHA + (1.0 - ALPHA) * prob_bins
    new_pb = new_pb / (jnp.sum(new_pb) + 1e-10) * NUM_PROB_BINS
    pb_out[...] = new_pb

    cls = jnp.sum(cnt_ref[...], axis=0)[None, :C]
    classes = cls / (jnp.sum(cls) + 1e-10) * C
    new_cls = ce_ref[...] * ALPHA + (1.0 - ALPHA) * classes
    new_cls = new_cls / (jnp.sum(new_cls) + 1e-10) * C
    cls_out[...] = new_cls


@jax.jit
def kernel(pred, target, classes_ema, prob_bins_ema):
    pred_flat = pred.reshape(N * C)
    tgt3 = target.reshape(NW, RQ, 128)
    zeros = jnp.zeros((C_PAD,), jnp.float32)
    ones = jnp.ones((RQ, 128), jnp.float32)
    ce2 = classes_ema.reshape(1, C)
    pb2 = jnp.pad(prob_bins_ema, (0, PB_PAD - NUM_PROB_BINS)).reshape(1, PB_PAD)

    tval, cet, cnt = _sc_gather(pred_flat, tgt3, classes_ema, zeros, ones)

    m, s = pl.pallas_call(
        _rowstats_body,
        grid=(NB,),
        in_specs=[pl.BlockSpec((ROWS, C), lambda i: (i, 0))],
        out_specs=[
            pl.BlockSpec((ROWS, 1), lambda i: (i, 0)),
            pl.BlockSpec((ROWS, 1), lambda i: (i, 0)),
        ],
        out_shape=[
            jax.ShapeDtypeStruct((N, 1), jnp.float32),
            jax.ShapeDtypeStruct((N, 1), jnp.float32),
        ],
        compiler_params=pltpu.CompilerParams(
            dimension_semantics=("arbitrary",),
        ),
    )(pred)

    loss_o, pb_o, cls_o = pl.pallas_call(
        _final_body,
        out_shape=[
            jax.ShapeDtypeStruct((1, PB_PAD), jnp.float32),
            jax.ShapeDtypeStruct((1, PB_PAD), jnp.float32),
            jax.ShapeDtypeStruct((1, C), jnp.float32),
        ],
    )(m.reshape(128, 128), s.reshape(128, 128),
      tval.reshape(128, 128), cet.reshape(128, 128),
      cnt, ce2, pb2)

    return loss_o[0, 0], pb_o[0, :NUM_PROB_BINS], cls_o[0, :]


# speedup vs baseline: 1.8091x; 1.8091x over previous
import functools

import jax
import jax.numpy as jnp
from jax.experimental import pallas as pl
from jax.experimental.pallas import tpu as pltpu

N = 16384
C = 1000
NUM_PROB_BINS = 10
ALPHA = 0.99
ROWS = 1024
NB = N // ROWS
PB_PAD = 128


def _body(pred_ref, tgt_ref, ce_ref, pb_ref,
          loss_out, pb_out, cls_out,
          loss_acc, hist_acc, cls_acc):
    i = pl.program_id(0)

    @pl.when(i == 0)
    def _init():
        loss_acc[0, 0] = 0.0
        hist_acc[...] = jnp.zeros_like(hist_acc)
        cls_acc[...] = jnp.zeros_like(cls_acc)

    x = pred_ref[...]
    tgt = tgt_ref[...]

    m = jnp.max(x, axis=1, keepdims=True)
    e = jnp.exp(x - m)
    s = jnp.sum(e, axis=1, keepdims=True)

    cls_iota = jax.lax.broadcasted_iota(jnp.int32, (ROWS, C), 1)
    onehot = (cls_iota == tgt).astype(jnp.float32)
    t_val = jnp.sum(x * onehot, axis=1, keepdims=True)
    ce_t = jnp.sum(ce_ref[...] * onehot, axis=1, keepdims=True)

    lse = m + jnp.log(s)
    loss = lse - t_val
    p = jnp.exp(t_val - m) / s
    p = jnp.clip(p, 1e-06, 1.0 - 1e-06)

    bin_idx = jnp.clip(jnp.floor(p * NUM_PROB_BINS - 1e-06).astype(jnp.int32),
                       0, NUM_PROB_BINS - 1)
    lane_iota = jax.lax.broadcasted_iota(jnp.int32, (ROWS, PB_PAD), 1)
    pb_t = jnp.sum(jnp.where(lane_iota == bin_idx, pb_ref[...], 0.0),
                   axis=1, keepdims=True)

    w = jnp.sqrt(ce_t * pb_t + 1e-10)
    loss_acc[0, 0] += jnp.sum(loss / w)

    hist_idx = jnp.clip(jnp.floor(p * NUM_PROB_BINS).astype(jnp.int32),
                        0, NUM_PROB_BINS - 1)
    hist_part = jnp.sum((lane_iota == hist_idx).astype(jnp.float32), axis=0)
    hist_acc[...] += hist_part[None, :]

    cls_acc[...] += jnp.sum(onehot, axis=0)[None, :]

    @pl.when(i == NB - 1)
    def _finalize():
        loss_out[...] = jnp.full((1, PB_PAD), loss_acc[0, 0] / N)

        hist = hist_acc[...]
        prob_bins = hist / (jnp.sum(hist) + 1e-10) * NUM_PROB_BINS
        new_pb = pb_ref[...] * ALPHA + (1.0 - ALPHA) * prob_bins
        new_pb = new_pb / (jnp.sum(new_pb) + 1e-10) * NUM_PROB_BINS
        pb_out[...] = new_pb

        cls = cls_acc[...]
        classes = cls / (jnp.sum(cls) + 1e-10) * C
        new_cls = ce_ref[...] * ALPHA + (1.0 - ALPHA) * classes
        new_cls = new_cls / (jnp.sum(new_cls) + 1e-10) * C
        cls_out[...] = new_cls


@functools.partial(jax.jit, static_argnames=())
def kernel(pred, target, classes_ema, prob_bins_ema):
    tgt2 = target.reshape(N, 1)
    ce2 = classes_ema.reshape(1, C)
    pb2 = jnp.pad(prob_bins_ema, (0, PB_PAD - NUM_PROB_BINS)).reshape(1, PB_PAD)

    loss_o, pb_o, cls_o = pl.pallas_call(
        _body,
        grid=(NB,),
        in_specs=[
            pl.BlockSpec((ROWS, C), lambda i: (i, 0)),
            pl.BlockSpec((ROWS, 1), lambda i: (i, 0)),
            pl.BlockSpec((1, C), lambda i: (0, 0)),
            pl.BlockSpec((1, PB_PAD), lambda i: (0, 0)),
        ],
        out_specs=[
            pl.BlockSpec((1, PB_PAD), lambda i: (0, 0)),
            pl.BlockSpec((1, PB_PAD), lambda i: (0, 0)),
            pl.BlockSpec((1, C), lambda i: (0, 0)),
        ],
        out_shape=[
            jax.ShapeDtypeStruct((1, PB_PAD), jnp.float32),
            jax.ShapeDtypeStruct((1, PB_PAD), jnp.float32),
            jax.ShapeDtypeStruct((1, C), jnp.float32),
        ],
        scratch_shapes=[
            pltpu.SMEM((1, 1), jnp.float32),
            pltpu.VMEM((1, PB_PAD), jnp.float32),
            pltpu.VMEM((1, C), jnp.float32),
        ],
        compiler_params=pltpu.CompilerParams(
            dimension_semantics=("arbitrary",),
        ),
    )(pred, tgt2, ce2, pb2)

    return loss_o[0, 0], pb_o[0, :NUM_PROB_BINS], cls_o[0, :]
